# baseline (speedup 1.0000x reference)
import functools

import jax
import jax.numpy as jnp
from jax import lax
from jax.experimental import pallas as pl
from jax.experimental.pallas import tpu as pltpu
from jax.experimental.pallas import tpu_sc as plsc

N = 10000
E = 320000
D = 128
NC = 2
NS = 16
EPT = E // NS
CHUNK = 128
NCH = 160
EPAD = NCH * CHUNK
NIB = 8
ROWS_PT = N // NS
RBLK = 1000
GRID = N // RBLK
HPAD = ((N + 1 + 15) // 16) * 16


_sc_mesh = plsc.VectorSubcoreMesh(core_axis_name="c", subcore_axis_name="s")


@functools.partial(
    pl.kernel,
    out_type=jax.ShapeDtypeStruct((NC, NS, HPAD), jnp.float32),
    mesh=_sc_mesh,
    scratch_types=[
        pltpu.VMEM((EPAD,), jnp.int32),
        pltpu.VMEM((HPAD,), jnp.float32),
    ],
    compiler_params=pltpu.CompilerParams(needs_layout_passes=False),
)
def _sc_degree(cols_hbm, hist_hbm, col_v, hist_v):
    c = lax.axis_index("c")
    s = lax.axis_index("s")
    pltpu.sync_copy(cols_hbm.at[c, s], col_v)

    zeros16 = jnp.zeros((16,), jnp.float32)

    def zbody(i, _):
        hist_v[pl.ds(i * 16, 16)] = zeros16
        return ()

    lax.fori_loop(0, HPAD // 16, zbody, (), unroll=8)

    ones16 = jnp.ones((16,), jnp.float32)

    def hbody(i, _):
        idx = col_v[pl.ds(i * 16, 16)]
        plsc.addupdate_scatter(hist_v, [idx], ones16)
        return ()

    lax.fori_loop(0, EPAD // 16, hbody, (), unroll=8)
    pltpu.sync_copy(hist_v, hist_hbm.at[c, s])


@functools.partial(
    pl.kernel,
    out_type=jax.ShapeDtypeStruct((NC, N, D), jnp.float32),
    mesh=_sc_mesh,
    scratch_types=[
        pltpu.VMEM((NIB, 2, CHUNK), jnp.int32),
        pltpu.VMEM((CHUNK, D), jnp.float32),
        pltpu.VMEM((CHUNK, D), jnp.float32),
        pltpu.VMEM_SHARED((N + 8, D), jnp.float32),
        pltpu.SemaphoreType.DMA((NIB,)),
        pltpu.SemaphoreType.DMA,
        pltpu.SemaphoreType.DMA,
        pltpu.SemaphoreType.DMA,
        pltpu.SemaphoreType.DMA,
    ],
)
def _sc_scatter(g_hbm, idx_hbm, acc_hbm,
                ibuf, buf0, buf1, acc_sh, semi, semg0, semg1, sema0, sema1):
    c = lax.axis_index("c")
    s = lax.axis_index("s")
    gflat = g_hbm.at[c]
    myidx = idx_hbm.at[c, s]

    @pl.when(s < NS - 1)
    def _():
        pltpu.sync_copy(gflat.at[pl.ds(s * 640, 640)],
                        acc_sh.at[pl.ds(s * 640, 640)])

    @pl.when(s == NS - 1)
    def _():
        pltpu.sync_copy(gflat.at[pl.ds(9600, 400)],
                        acc_sh.at[pl.ds(9600, 400)])

    plsc.subcore_barrier()

    bufs = (buf0, buf1)
    semg = (semg0, semg1)
    sema = (sema0, sema1)

    def fire_idx(j, i):
        pltpu.async_copy(myidx.at[j], ibuf.at[i % NIB], semi.at[i % NIB])

    def wait_idx(j, i):
        pltpu.make_async_copy(
            myidx.at[j], ibuf.at[i % NIB], semi.at[i % NIB]).wait()

    def fire_g(j, i):
        pltpu.async_copy(gflat.at[ibuf.at[i % NIB, 0]], bufs[i % 2],
                         semg[i % 2])

    def wait_g(j, i):
        pltpu.make_async_copy(gflat.at[ibuf.at[i % NIB, 0]], bufs[i % 2],
                              semg[i % 2]).wait()

    def fire_a(j, i):
        pltpu.async_copy(bufs[i % 2], acc_sh.at[ibuf.at[i % NIB, 1]],
                         sema[i % 2], add=True)

    def wait_a(j, i):
        pltpu.make_async_copy(bufs[i % 2], acc_sh.at[ibuf.at[i % NIB, 1]],
                              sema[i % 2]).wait()

    def step(j, i, first=False, fetch=True, advance=True):
        wait_g(j, i)
        fire_a(j, i)
        if advance:
            wait_idx(j + 1, i + 1)
            if not first:
                wait_a(j - 1, i + 1)
            fire_g(j + 1, i + 1)
        if fetch:
            fire_idx(j + 6, i + 6)

    pltpu.sync_copy(myidx.at[0], ibuf.at[0])
    for j in range(1, 6):
        fire_idx(j, j)
    fire_g(0, 0)
    for i in range(8):
        step(i, i, first=(i == 0))

    def body(k, _):
        j0 = k * 8
        for i in range(8):
            step(j0 + i, i)
        return ()

    lax.fori_loop(1, (NCH // 8) - 1, body, ())

    j0 = NCH - 8
    for i in range(8):
        j = j0 + i
        step(j, i, fetch=(j + 6 <= NCH - 1), advance=(j < NCH - 1))
    wait_a(NCH - 2, 6)
    wait_a(NCH - 1, 7)

    plsc.subcore_barrier()

    @pl.when(s < NS - 1)
    def _():
        pltpu.sync_copy(acc_sh.at[pl.ds(s * 640, 640)],
                        acc_hbm.at[c].at[pl.ds(s * 640, 640)])

    @pl.when(s == NS - 1)
    def _():
        pltpu.sync_copy(acc_sh.at[pl.ds(9600, 400)],
                        acc_hbm.at[c].at[pl.ds(9600, 400)])


def _tc_dinv_body(hist_ref, dinv_ref):
    deg = jnp.sum(hist_ref[...], axis=1) + 1.0
    dinv_ref[...] = lax.rsqrt(deg)[:, :N, None]


_tc_dinv = pl.pallas_call(
    _tc_dinv_body,
    out_shape=jax.ShapeDtypeStruct((NC, N, 1), jnp.float32),
)


def _tc_layer1_body(x_ref, w0_ref, w1_ref, dinv_ref, g_ref):
    dinv = dinv_ref[...]
    xb = x_ref[...]
    h0 = jnp.dot(xb, w0_ref[...], preferred_element_type=jnp.float32)
    h1 = jnp.dot(xb, w1_ref[...], preferred_element_type=jnp.float32)
    g_ref[0] = h0 * dinv[0]
    g_ref[1] = h1 * dinv[1]


def _tc_layer2_body(acc_ref, dinv_ref, b1_ref, w0_ref, w1_ref, g_ref):
    dinv = dinv_ref[...]
    h = jax.nn.relu(acc_ref[0] * dinv[0] + b1_ref[0]
                    + acc_ref[1] * dinv[1] + b1_ref[1])
    h0 = jnp.dot(h, w0_ref[...], preferred_element_type=jnp.float32)
    h1 = jnp.dot(h, w1_ref[...], preferred_element_type=jnp.float32)
    g_ref[0] = h0 * dinv[0]
    g_ref[1] = h1 * dinv[1]


def _tc_final_body(acc_ref, dinv_ref, b2_ref, out_ref):
    dinv = dinv_ref[...]
    out_ref[...] = (acc_ref[0] * dinv[0] + b2_ref[0]
                    + acc_ref[1] * dinv[1] + b2_ref[1])


_w_spec = pl.BlockSpec((D, D), lambda i: (0, 0))
_b_spec = pl.BlockSpec((NC, 1, D), lambda i: (0, 0, 0))
_g_spec = pl.BlockSpec((NC, RBLK, D), lambda i: (0, i, 0))
_dinv_spec = pl.BlockSpec((NC, RBLK, 1), lambda i: (0, i, 0))
_x_spec = pl.BlockSpec((RBLK, D), lambda i: (i, 0))

_tc_layer1 = pl.pallas_call(
    _tc_layer1_body,
    grid=(GRID,),
    in_specs=[_x_spec, _w_spec, _w_spec, _dinv_spec],
    out_specs=_g_spec,
    out_shape=jax.ShapeDtypeStruct((NC, N, D), jnp.float32),
)

_tc_layer2 = pl.pallas_call(
    _tc_layer2_body,
    grid=(GRID,),
    in_specs=[_g_spec, _dinv_spec, _b_spec, _w_spec, _w_spec],
    out_specs=_g_spec,
    out_shape=jax.ShapeDtypeStruct((NC, N, D), jnp.float32),
)

_tc_final = pl.pallas_call(
    _tc_final_body,
    grid=(GRID,),
    in_specs=[_g_spec, _dinv_spec, _b_spec],
    out_specs=_x_spec,
    out_shape=jax.ShapeDtypeStruct((N, D), jnp.float32),
)


def _prep_indices(ei):
    r = ei[0].astype(jnp.int32).reshape(NS, EPT)
    c = ei[1].astype(jnp.int32).reshape(NS, EPT)
    pad = ((0, 0), (0, EPAD - EPT))
    r = jnp.pad(r, pad, constant_values=0).reshape(NS, NCH, CHUNK)
    c = jnp.pad(c, pad, constant_values=N).reshape(NS, NCH, CHUNK)
    return jnp.stack([r, c], axis=2), c.reshape(NS, EPAD)


@jax.jit
def kernel(x, edge_index_0, edge_index_1,
           W1_0, b1_0, W1_1, b1_1, W2_0, b2_0, W2_1, b2_1):
    i0, c0 = _prep_indices(edge_index_0)
    i1, c1 = _prep_indices(edge_index_1)
    idx = jnp.stack([i0, i1])
    cols_flat = jnp.stack([c0, c1])

    hist = _sc_degree(cols_flat)
    b1 = jnp.stack([b1_0, b1_1]).reshape(NC, 1, D)
    b2 = jnp.stack([b2_0, b2_1]).reshape(NC, 1, D)

    dinv = _tc_dinv(hist)
    g1 = _tc_layer1(x, W1_0, W1_1, dinv)
    acc1 = _sc_scatter(g1, idx)
    g2 = _tc_layer2(acc1, dinv, b1, W2_0, W2_1)
    acc2 = _sc_scatter(g2, idx)
    return _tc_final(acc2, dinv, b2)


# speedup vs baseline: 1.4987x; 1.4987x over previous
import functools

import jax
import jax.numpy as jnp
from jax import lax
from jax.experimental import pallas as pl
from jax.experimental.pallas import tpu as pltpu
from jax.experimental.pallas import tpu_sc as plsc

N = 10000
E = 320000
D = 128
NC = 2
NS = 16
EPT = E // NS
CHUNK = 128
NCH = (EPT + CHUNK - 1) // CHUNK
EPAD = NCH * CHUNK
ROWS_PT = N // NS
RBLK = 1000
GRID = N // RBLK
HPAD = ((N + 1 + 15) // 16) * 16


_sc_mesh = plsc.VectorSubcoreMesh(core_axis_name="c", subcore_axis_name="s")


@functools.partial(
    pl.kernel,
    out_type=jax.ShapeDtypeStruct((NC, NS, HPAD), jnp.float32),
    mesh=_sc_mesh,
    scratch_types=[
        pltpu.VMEM((EPAD,), jnp.int32),
        pltpu.VMEM((HPAD,), jnp.float32),
    ],
    compiler_params=pltpu.CompilerParams(needs_layout_passes=False),
)
def _sc_degree(cols_hbm, hist_hbm, col_v, hist_v):
    c = lax.axis_index("c")
    s = lax.axis_index("s")
    pltpu.sync_copy(cols_hbm.at[c, s], col_v)

    zeros16 = jnp.zeros((16,), jnp.float32)

    def zbody(i, _):
        hist_v[pl.ds(i * 16, 16)] = zeros16
        return ()

    lax.fori_loop(0, HPAD // 16, zbody, (), unroll=8)

    ones16 = jnp.ones((16,), jnp.float32)

    def hbody(i, _):
        idx = col_v[pl.ds(i * 16, 16)]
        plsc.addupdate_scatter(hist_v, [idx], ones16)
        return ()

    lax.fori_loop(0, EPAD // 16, hbody, (), unroll=8)
    pltpu.sync_copy(hist_v, hist_hbm.at[c, s])


@functools.partial(
    pl.kernel,
    out_type=jax.ShapeDtypeStruct((NC, N, D), jnp.float32),
    mesh=_sc_mesh,
    scratch_types=[
        pltpu.VMEM((2, CHUNK), jnp.int32),
        pltpu.VMEM((2, CHUNK), jnp.int32),
        pltpu.VMEM((CHUNK, D), jnp.float32),
        pltpu.VMEM((CHUNK, D), jnp.float32),
        pltpu.VMEM_SHARED((N + 8, D), jnp.float32),
        pltpu.SemaphoreType.DMA,
        pltpu.SemaphoreType.DMA,
        pltpu.SemaphoreType.DMA,
        pltpu.SemaphoreType.DMA,
    ],
)
def _sc_scatter(g_hbm, idx_hbm, acc_hbm,
                ibuf0, ibuf1, buf0, buf1, acc_sh, semi0, semi1, semg0, semg1):
    c = lax.axis_index("c")
    s = lax.axis_index("s")
    gflat = g_hbm.at[c]
    myidx = idx_hbm.at[c, s]

    @pl.when(s < NS - 1)
    def _():
        pltpu.sync_copy(gflat.at[pl.ds(s * 640, 640)],
                        acc_sh.at[pl.ds(s * 640, 640)])

    @pl.when(s == NS - 1)
    def _():
        pltpu.sync_copy(gflat.at[pl.ds(9600, 400)],
                        acc_sh.at[pl.ds(9600, 400)])

    plsc.subcore_barrier()

    pltpu.sync_copy(myidx.at[0], ibuf0)
    pltpu.async_copy(myidx.at[1], ibuf1, semi1)
    pltpu.async_copy(gflat.at[ibuf0.at[0]], buf0, semg0)

    def body(jj, _):
        j0 = 2 * jj
        pltpu.make_async_copy(gflat.at[ibuf0.at[0]], buf0, semg0).wait()
        pltpu.make_async_copy(myidx.at[j0 + 1], ibuf1, semi1).wait()
        pltpu.async_copy(gflat.at[ibuf1.at[0]], buf1, semg1)
        pltpu.sync_copy(buf0, acc_sh.at[ibuf0.at[1]], add=True)
        pltpu.async_copy(myidx.at[j0 + 2], ibuf0, semi0)
        pltpu.make_async_copy(gflat.at[ibuf1.at[0]], buf1, semg1).wait()
        pltpu.sync_copy(buf1, acc_sh.at[ibuf1.at[1]], add=True)
        pltpu.make_async_copy(myidx.at[j0 + 2], ibuf0, semi0).wait()
        pltpu.async_copy(gflat.at[ibuf0.at[0]], buf0, semg0)
        pltpu.async_copy(myidx.at[j0 + 3], ibuf1, semi1)
        return ()

    lax.fori_loop(0, (NCH - 3) // 2, body, ())
    pltpu.make_async_copy(gflat.at[ibuf0.at[0]], buf0, semg0).wait()
    pltpu.make_async_copy(myidx.at[NCH - 2], ibuf1, semi1).wait()
    pltpu.async_copy(gflat.at[ibuf1.at[0]], buf1, semg1)
    pltpu.sync_copy(buf0, acc_sh.at[ibuf0.at[1]], add=True)
    pltpu.async_copy(myidx.at[NCH - 1], ibuf0, semi0)
    pltpu.make_async_copy(gflat.at[ibuf1.at[0]], buf1, semg1).wait()
    pltpu.sync_copy(buf1, acc_sh.at[ibuf1.at[1]], add=True)
    pltpu.make_async_copy(myidx.at[NCH - 1], ibuf0, semi0).wait()
    pltpu.async_copy(gflat.at[ibuf0.at[0]], buf0, semg0)
    pltpu.make_async_copy(gflat.at[ibuf0.at[0]], buf0, semg0).wait()
    pltpu.sync_copy(buf0, acc_sh.at[ibuf0.at[1]], add=True)

    plsc.subcore_barrier()

    @pl.when(s < NS - 1)
    def _():
        pltpu.sync_copy(acc_sh.at[pl.ds(s * 640, 640)],
                        acc_hbm.at[c].at[pl.ds(s * 640, 640)])

    @pl.when(s == NS - 1)
    def _():
        pltpu.sync_copy(acc_sh.at[pl.ds(9600, 400)],
                        acc_hbm.at[c].at[pl.ds(9600, 400)])


def _tc_dinv_body(hist_ref, dinv_ref):
    deg = jnp.sum(hist_ref[...], axis=1) + 1.0
    dinv_ref[...] = lax.rsqrt(deg)[:, :N, None]


_tc_dinv = pl.pallas_call(
    _tc_dinv_body,
    out_shape=jax.ShapeDtypeStruct((NC, N, 1), jnp.float32),
)


def _tc_layer1_body(x_ref, w0_ref, w1_ref, dinv_ref, g_ref):
    dinv = dinv_ref[...]
    xb = x_ref[...]
    h0 = jnp.dot(xb, w0_ref[...], preferred_element_type=jnp.float32)
    h1 = jnp.dot(xb, w1_ref[...], preferred_element_type=jnp.float32)
    g_ref[0] = h0 * dinv[0]
    g_ref[1] = h1 * dinv[1]


def _tc_layer2_body(acc_ref, dinv_ref, b1_ref, w0_ref, w1_ref, g_ref):
    dinv = dinv_ref[...]
    h = jax.nn.relu(acc_ref[0] * dinv[0] + b1_ref[0]
                    + acc_ref[1] * dinv[1] + b1_ref[1])
    h0 = jnp.dot(h, w0_ref[...], preferred_element_type=jnp.float32)
    h1 = jnp.dot(h, w1_ref[...], preferred_element_type=jnp.float32)
    g_ref[0] = h0 * dinv[0]
    g_ref[1] = h1 * dinv[1]


def _tc_final_body(acc_ref, dinv_ref, b2_ref, out_ref):
    dinv = dinv_ref[...]
    out_ref[...] = (acc_ref[0] * dinv[0] + b2_ref[0]
                    + acc_ref[1] * dinv[1] + b2_ref[1])


_w_spec = pl.BlockSpec((D, D), lambda i: (0, 0))
_b_spec = pl.BlockSpec((NC, 1, D), lambda i: (0, 0, 0))
_g_spec = pl.BlockSpec((NC, RBLK, D), lambda i: (0, i, 0))
_dinv_spec = pl.BlockSpec((NC, RBLK, 1), lambda i: (0, i, 0))
_x_spec = pl.BlockSpec((RBLK, D), lambda i: (i, 0))

_tc_layer1 = pl.pallas_call(
    _tc_layer1_body,
    grid=(GRID,),
    in_specs=[_x_spec, _w_spec, _w_spec, _dinv_spec],
    out_specs=_g_spec,
    out_shape=jax.ShapeDtypeStruct((NC, N, D), jnp.float32),
)

_tc_layer2 = pl.pallas_call(
    _tc_layer2_body,
    grid=(GRID,),
    in_specs=[_g_spec, _dinv_spec, _b_spec, _w_spec, _w_spec],
    out_specs=_g_spec,
    out_shape=jax.ShapeDtypeStruct((NC, N, D), jnp.float32),
)

_tc_final = pl.pallas_call(
    _tc_final_body,
    grid=(GRID,),
    in_specs=[_g_spec, _dinv_spec, _b_spec],
    out_specs=_x_spec,
    out_shape=jax.ShapeDtypeStruct((N, D), jnp.float32),
)


def _prep_indices(ei):
    r = ei[0].astype(jnp.int32).reshape(NS, EPT)
    c = ei[1].astype(jnp.int32).reshape(NS, EPT)
    pad = ((0, 0), (0, EPAD - EPT))
    r = jnp.pad(r, pad, constant_values=0).reshape(NS, NCH, CHUNK)
    c = jnp.pad(c, pad, constant_values=N).reshape(NS, NCH, CHUNK)
    return jnp.stack([r, c], axis=2), c.reshape(NS, EPAD)


@jax.jit
def kernel(x, edge_index_0, edge_index_1,
           W1_0, b1_0, W1_1, b1_1, W2_0, b2_0, W2_1, b2_1):
    i0, c0 = _prep_indices(edge_index_0)
    i1, c1 = _prep_indices(edge_index_1)
    idx = jnp.stack([i0, i1])
    cols_flat = jnp.stack([c0, c1])

    hist = _sc_degree(cols_flat)
    b1 = jnp.stack([b1_0, b1_1]).reshape(NC, 1, D)
    b2 = jnp.stack([b2_0, b2_1]).reshape(NC, 1, D)

    dinv = _tc_dinv(hist)
    g1 = _tc_layer1(x, W1_0, W1_1, dinv)
    acc1 = _sc_scatter(g1, idx)
    g2 = _tc_layer2(acc1, dinv, b1, W2_0, W2_1)
    acc2 = _sc_scatter(g2, idx)
    return _tc_final(acc2, dinv, b2)


# speedup vs baseline: 1.6832x; 1.1231x over previous
import functools

import jax
import jax.numpy as jnp
from jax import lax
from jax.experimental import pallas as pl
from jax.experimental.pallas import tpu as pltpu
from jax.experimental.pallas import tpu_sc as plsc

N = 10000
E = 320000
D = 128
NC = 2
NS = 16
EPT = E // NS
CHUNK = 128
NCH = EPT // CHUNK + 1
EPAD = NCH * CHUNK
RBLK = 1000
GRID = N // RBLK
HPAD = ((N + 1 + 15) // 16) * 16


_sc_mesh = plsc.VectorSubcoreMesh(core_axis_name="c", subcore_axis_name="s")


@functools.partial(
    pl.kernel,
    out_type=jax.ShapeDtypeStruct((NC, NS, HPAD), jnp.float32),
    mesh=_sc_mesh,
    scratch_types=[
        pltpu.VMEM((EPAD,), jnp.int32),
        pltpu.VMEM((HPAD,), jnp.float32),
    ],
    compiler_params=pltpu.CompilerParams(needs_layout_passes=False),
)
def _sc_degree(cols_hbm, hist_hbm, col_v, hist_v):
    c = lax.axis_index("c")
    s = lax.axis_index("s")
    pltpu.sync_copy(cols_hbm.at[c, s], col_v)

    zeros16 = jnp.zeros((16,), jnp.float32)

    def zbody(i, _):
        hist_v[pl.ds(i * 16, 16)] = zeros16
        return ()

    lax.fori_loop(0, HPAD // 16, zbody, (), unroll=8)

    ones16 = jnp.ones((16,), jnp.float32)

    def hbody(i, _):
        idx = col_v[pl.ds(i * 16, 16)]
        plsc.addupdate_scatter(hist_v, [idx], ones16)
        return ()

    lax.fori_loop(0, EPAD // 16, hbody, (), unroll=8)
    pltpu.sync_copy(hist_v, hist_hbm.at[c, s])


@functools.partial(
    pl.kernel,
    out_type=jax.ShapeDtypeStruct((NC, N, D), jnp.float32),
    mesh=_sc_mesh,
    scratch_types=[
        pltpu.VMEM((2, 2, CHUNK), jnp.int32),
        pltpu.VMEM((2, 2, CHUNK), jnp.int32),
        pltpu.VMEM((CHUNK, D), jnp.float32),
        pltpu.VMEM((CHUNK, D), jnp.float32),
        pltpu.VMEM_SHARED((N + 8, D), jnp.float32),
        pltpu.SemaphoreType.DMA,
        pltpu.SemaphoreType.DMA,
        pltpu.SemaphoreType.DMA,
        pltpu.SemaphoreType.DMA,
    ],
)
def _sc_scatter(g_hbm, idx_hbm, acc_hbm,
                ibufa, ibufb, buf0, buf1, acc_sh, semia, semib, semg0, semg1):
    c = lax.axis_index("c")
    s = lax.axis_index("s")
    gflat = g_hbm.at[c]
    myidx = idx_hbm.at[c, s]

    def idx_pair(j):
        return myidx.at[pl.ds(j, 2)]

    @pl.when(s < NS - 1)
    def _():
        pltpu.sync_copy(gflat.at[pl.ds(s * 640, 640)],
                        acc_sh.at[pl.ds(s * 640, 640)])

    @pl.when(s == NS - 1)
    def _():
        pltpu.sync_copy(gflat.at[pl.ds(9600, 400)],
                        acc_sh.at[pl.ds(9600, 400)])

    plsc.subcore_barrier()

    pltpu.sync_copy(idx_pair(0), ibufa)
    pltpu.async_copy(idx_pair(2), ibufb, semib)
    pltpu.async_copy(gflat.at[ibufa.at[0, 0]], buf0, semg0)

    def body(kk, _):
        j0 = 4 * kk
        pltpu.make_async_copy(gflat.at[ibufa.at[0, 0]], buf0, semg0).wait()
        pltpu.async_copy(gflat.at[ibufa.at[1, 0]], buf1, semg1)
        pltpu.sync_copy(buf0, acc_sh.at[ibufa.at[0, 1]], add=True)
        pltpu.make_async_copy(idx_pair(j0 + 2), ibufb, semib).wait()
        pltpu.make_async_copy(gflat.at[ibufa.at[1, 0]], buf1, semg1).wait()
        pltpu.async_copy(gflat.at[ibufb.at[0, 0]], buf0, semg0)
        pltpu.sync_copy(buf1, acc_sh.at[ibufa.at[1, 1]], add=True)
        pltpu.async_copy(idx_pair(j0 + 4), ibufa, semia)
        pltpu.make_async_copy(gflat.at[ibufb.at[0, 0]], buf0, semg0).wait()
        pltpu.async_copy(gflat.at[ibufb.at[1, 0]], buf1, semg1)
        pltpu.sync_copy(buf0, acc_sh.at[ibufb.at[0, 1]], add=True)
        pltpu.make_async_copy(idx_pair(j0 + 4), ibufa, semia).wait()
        pltpu.make_async_copy(gflat.at[ibufb.at[1, 0]], buf1, semg1).wait()
        pltpu.async_copy(gflat.at[ibufa.at[0, 0]], buf0, semg0)
        pltpu.sync_copy(buf1, acc_sh.at[ibufb.at[1, 1]], add=True)
        pltpu.async_copy(idx_pair(j0 + 6), ibufb, semib)
        return ()

    lax.fori_loop(0, (NCH - 5) // 4, body, ())

    pltpu.make_async_copy(gflat.at[ibufa.at[0, 0]], buf0, semg0).wait()
    pltpu.async_copy(gflat.at[ibufa.at[1, 0]], buf1, semg1)
    pltpu.sync_copy(buf0, acc_sh.at[ibufa.at[0, 1]], add=True)
    pltpu.make_async_copy(idx_pair(NCH - 3), ibufb, semib).wait()
    pltpu.make_async_copy(gflat.at[ibufa.at[1, 0]], buf1, semg1).wait()
    pltpu.async_copy(gflat.at[ibufb.at[0, 0]], buf0, semg0)
    pltpu.sync_copy(buf1, acc_sh.at[ibufa.at[1, 1]], add=True)
    pltpu.sync_copy(myidx.at[NCH - 1], ibufa.at[0])
    pltpu.make_async_copy(gflat.at[ibufb.at[0, 0]], buf0, semg0).wait()
    pltpu.async_copy(gflat.at[ibufb.at[1, 0]], buf1, semg1)
    pltpu.sync_copy(buf0, acc_sh.at[ibufb.at[0, 1]], add=True)
    pltpu.async_copy(gflat.at[ibufa.at[0, 0]], buf0, semg0)
    pltpu.make_async_copy(gflat.at[ibufb.at[1, 0]], buf1, semg1).wait()
    pltpu.sync_copy(buf1, acc_sh.at[ibufb.at[1, 1]], add=True)
    pltpu.make_async_copy(gflat.at[ibufa.at[0, 0]], buf0, semg0).wait()
    pltpu.sync_copy(buf0, acc_sh.at[ibufa.at[0, 1]], add=True)

    plsc.subcore_barrier()

    @pl.when(s < NS - 1)
    def _():
        pltpu.sync_copy(acc_sh.at[pl.ds(s * 640, 640)],
                        acc_hbm.at[c].at[pl.ds(s * 640, 640)])

    @pl.when(s == NS - 1)
    def _():
        pltpu.sync_copy(acc_sh.at[pl.ds(9600, 400)],
                        acc_hbm.at[c].at[pl.ds(9600, 400)])


def _tc_dinv_body(hist_ref, dinv_ref):
    deg = jnp.sum(hist_ref[...], axis=1) + 1.0
    dinv_ref[...] = lax.rsqrt(deg)[:, :N, None]


_tc_dinv = pl.pallas_call(
    _tc_dinv_body,
    out_shape=jax.ShapeDtypeStruct((NC, N, 1), jnp.float32),
)


def _tc_layer1_body(x_ref, w0_ref, w1_ref, dinv_ref, g_ref):
    dinv = dinv_ref[...]
    xb = x_ref[...]
    h0 = jnp.dot(xb, w0_ref[...], preferred_element_type=jnp.float32)
    h1 = jnp.dot(xb, w1_ref[...], preferred_element_type=jnp.float32)
    g_ref[0] = h0 * dinv[0]
    g_ref[1] = h1 * dinv[1]


def _tc_layer2_body(acc_ref, dinv_ref, b1_ref, w0_ref, w1_ref, g_ref):
    dinv = dinv_ref[...]
    h = jax.nn.relu(acc_ref[0] * dinv[0] + b1_ref[0]
                    + acc_ref[1] * dinv[1] + b1_ref[1])
    h0 = jnp.dot(h, w0_ref[...], preferred_element_type=jnp.float32)
    h1 = jnp.dot(h, w1_ref[...], preferred_element_type=jnp.float32)
    g_ref[0] = h0 * dinv[0]
    g_ref[1] = h1 * dinv[1]


def _tc_final_body(acc_ref, dinv_ref, b2_ref, out_ref):
    dinv = dinv_ref[...]
    out_ref[...] = (acc_ref[0] * dinv[0] + b2_ref[0]
                    + acc_ref[1] * dinv[1] + b2_ref[1])


_w_spec = pl.BlockSpec((D, D), lambda i: (0, 0))
_b_spec = pl.BlockSpec((NC, 1, D), lambda i: (0, 0, 0))
_g_spec = pl.BlockSpec((NC, RBLK, D), lambda i: (0, i, 0))
_dinv_spec = pl.BlockSpec((NC, RBLK, 1), lambda i: (0, i, 0))
_x_spec = pl.BlockSpec((RBLK, D), lambda i: (i, 0))

_tc_layer1 = pl.pallas_call(
    _tc_layer1_body,
    grid=(GRID,),
    in_specs=[_x_spec, _w_spec, _w_spec, _dinv_spec],
    out_specs=_g_spec,
    out_shape=jax.ShapeDtypeStruct((NC, N, D), jnp.float32),
)

_tc_layer2 = pl.pallas_call(
    _tc_layer2_body,
    grid=(GRID,),
    in_specs=[_g_spec, _dinv_spec, _b_spec, _w_spec, _w_spec],
    out_specs=_g_spec,
    out_shape=jax.ShapeDtypeStruct((NC, N, D), jnp.float32),
)

_tc_final = pl.pallas_call(
    _tc_final_body,
    grid=(GRID,),
    in_specs=[_g_spec, _dinv_spec, _b_spec],
    out_specs=_x_spec,
    out_shape=jax.ShapeDtypeStruct((N, D), jnp.float32),
)


def _prep_indices(ei):
    r = ei[0].astype(jnp.int32).reshape(NS, EPT)
    c = ei[1].astype(jnp.int32).reshape(NS, EPT)
    pad = ((0, 0), (0, EPAD - EPT))
    r = jnp.pad(r, pad, constant_values=0).reshape(NS, NCH, CHUNK)
    c = jnp.pad(c, pad, constant_values=N).reshape(NS, NCH, CHUNK)
    return jnp.stack([r, c], axis=2), c.reshape(NS, EPAD)


@jax.jit
def kernel(x, edge_index_0, edge_index_1,
           W1_0, b1_0, W1_1, b1_1, W2_0, b2_0, W2_1, b2_1):
    i0, c0 = _prep_indices(edge_index_0)
    i1, c1 = _prep_indices(edge_index_1)
    idx = jnp.stack([i0, i1])
    cols_flat = jnp.stack([c0, c1])

    hist = _sc_degree(cols_flat)
    b1 = jnp.stack([b1_0, b1_1]).reshape(NC, 1, D)
    b2 = jnp.stack([b2_0, b2_1]).reshape(NC, 1, D)

    dinv = _tc_dinv(hist)
    g1 = _tc_layer1(x, W1_0, W1_1, dinv)
    acc1 = _sc_scatter(g1, idx)
    g2 = _tc_layer2(acc1, dinv, b1, W2_0, W2_1)
    acc2 = _sc_scatter(g2, idx)
    return _tc_final(acc2, dinv, b2)


# speedup vs baseline: 1.6911x; 1.0047x over previous
import functools

import jax
import jax.numpy as jnp
from jax import lax
from jax.experimental import pallas as pl
from jax.experimental.pallas import tpu as pltpu
from jax.experimental.pallas import tpu_sc as plsc

N = 10000
E = 320000
D = 128
NC = 2
NS = 16
EPT = E // NS
CHUNK = 128
NCH = EPT // CHUNK + 1
EPAD = NCH * CHUNK
RBLK = 1000
GRID = N // RBLK
HPAD = ((N + 1 + 15) // 16) * 16


_sc_mesh = plsc.VectorSubcoreMesh(core_axis_name="c", subcore_axis_name="s")


SLAB = 19968

@functools.partial(
    pl.kernel,
    out_type=jax.ShapeDtypeStruct((NC, NS, HPAD), jnp.float32),
    mesh=_sc_mesh,
    scratch_types=[
        pltpu.VMEM((SLAB,), jnp.int32),
        pltpu.VMEM((HPAD,), jnp.float32),
    ],
    compiler_params=pltpu.CompilerParams(needs_layout_passes=False),
)
def _sc_degree(dst_hbm, hist_hbm, col_v, hist_v):
    c = lax.axis_index("c")
    s = lax.axis_index("s")
    base = pl.multiple_of(s * SLAB, 128)
    pltpu.sync_copy(dst_hbm.at[c, pl.ds(base, SLAB)], col_v)

    zeros16 = jnp.zeros((16,), jnp.float32)

    def zbody(i, _):
        hist_v[pl.ds(i * 16, 16)] = zeros16
        return ()

    lax.fori_loop(0, HPAD // 16, zbody, (), unroll=8)

    ones16 = jnp.ones((16,), jnp.float32)

    def hbody(i, _):
        idx = col_v[pl.ds(i * 16, 16)]
        plsc.addupdate_scatter(hist_v, [idx], ones16)
        return ()

    lax.fori_loop(0, SLAB // 16, hbody, (), unroll=8)

    @pl.when(s < 4)
    def _():
        xb = pl.multiple_of(NS * SLAB + s * CHUNK, 128)
        pltpu.sync_copy(dst_hbm.at[c, pl.ds(xb, CHUNK)],
                        col_v.at[pl.ds(0, CHUNK)])

        def xbody(i, _):
            idx = col_v[pl.ds(i * 16, 16)]
            plsc.addupdate_scatter(hist_v, [idx], ones16)
            return ()

        lax.fori_loop(0, CHUNK // 16, xbody, ())

    pltpu.sync_copy(hist_v, hist_hbm.at[c, s])


@functools.partial(
    pl.kernel,
    out_type=jax.ShapeDtypeStruct((NC, N, D), jnp.float32),
    mesh=_sc_mesh,
    scratch_types=[
        pltpu.VMEM((2, 2, CHUNK), jnp.int32),
        pltpu.VMEM((2, 2, CHUNK), jnp.int32),
        pltpu.VMEM((CHUNK, D), jnp.float32),
        pltpu.VMEM((CHUNK, D), jnp.float32),
        pltpu.VMEM_SHARED((N + 8, D), jnp.float32),
        pltpu.SemaphoreType.DMA,
        pltpu.SemaphoreType.DMA,
        pltpu.SemaphoreType.DMA,
        pltpu.SemaphoreType.DMA,
    ],
)
def _sc_scatter(g_hbm, idx_hbm, acc_hbm,
                ibufa, ibufb, buf0, buf1, acc_sh, semia, semib, semg0, semg1):
    c = lax.axis_index("c")
    s = lax.axis_index("s")
    gflat = g_hbm.at[c]
    myidx = idx_hbm.at[c, s]

    def idx_pair(j):
        return myidx.at[pl.ds(j, 2)]

    @pl.when(s < NS - 1)
    def _():
        pltpu.sync_copy(gflat.at[pl.ds(s * 640, 640)],
                        acc_sh.at[pl.ds(s * 640, 640)])

    @pl.when(s == NS - 1)
    def _():
        pltpu.sync_copy(gflat.at[pl.ds(9600, 400)],
                        acc_sh.at[pl.ds(9600, 400)])

    plsc.subcore_barrier()

    pltpu.sync_copy(idx_pair(0), ibufa)
    pltpu.async_copy(idx_pair(2), ibufb, semib)
    pltpu.async_copy(gflat.at[ibufa.at[0, 0]], buf0, semg0)

    def body(kk, _):
        j0 = 4 * kk
        pltpu.make_async_copy(gflat.at[ibufa.at[0, 0]], buf0, semg0).wait()
        pltpu.async_copy(gflat.at[ibufa.at[1, 0]], buf1, semg1)
        pltpu.sync_copy(buf0, acc_sh.at[ibufa.at[0, 1]], add=True)
        pltpu.make_async_copy(idx_pair(j0 + 2), ibufb, semib).wait()
        pltpu.make_async_copy(gflat.at[ibufa.at[1, 0]], buf1, semg1).wait()
        pltpu.async_copy(gflat.at[ibufb.at[0, 0]], buf0, semg0)
        pltpu.sync_copy(buf1, acc_sh.at[ibufa.at[1, 1]], add=True)
        pltpu.async_copy(idx_pair(j0 + 4), ibufa, semia)
        pltpu.make_async_copy(gflat.at[ibufb.at[0, 0]], buf0, semg0).wait()
        pltpu.async_copy(gflat.at[ibufb.at[1, 0]], buf1, semg1)
        pltpu.sync_copy(buf0, acc_sh.at[ibufb.at[0, 1]], add=True)
        pltpu.make_async_copy(idx_pair(j0 + 4), ibufa, semia).wait()
        pltpu.make_async_copy(gflat.at[ibufb.at[1, 0]], buf1, semg1).wait()
        pltpu.async_copy(gflat.at[ibufa.at[0, 0]], buf0, semg0)
        pltpu.sync_copy(buf1, acc_sh.at[ibufb.at[1, 1]], add=True)
        pltpu.async_copy(idx_pair(j0 + 6), ibufb, semib)
        return ()

    lax.fori_loop(0, (NCH - 5) // 4, body, ())

    pltpu.make_async_copy(gflat.at[ibufa.at[0, 0]], buf0, semg0).wait()
    pltpu.async_copy(gflat.at[ibufa.at[1, 0]], buf1, semg1)
    pltpu.sync_copy(buf0, acc_sh.at[ibufa.at[0, 1]], add=True)
    pltpu.make_async_copy(idx_pair(NCH - 3), ibufb, semib).wait()
    pltpu.make_async_copy(gflat.at[ibufa.at[1, 0]], buf1, semg1).wait()
    pltpu.async_copy(gflat.at[ibufb.at[0, 0]], buf0, semg0)
    pltpu.sync_copy(buf1, acc_sh.at[ibufa.at[1, 1]], add=True)
    pltpu.sync_copy(myidx.at[NCH - 1], ibufa.at[0])
    pltpu.make_async_copy(gflat.at[ibufb.at[0, 0]], buf0, semg0).wait()
    pltpu.async_copy(gflat.at[ibufb.at[1, 0]], buf1, semg1)
    pltpu.sync_copy(buf0, acc_sh.at[ibufb.at[0, 1]], add=True)
    pltpu.async_copy(gflat.at[ibufa.at[0, 0]], buf0, semg0)
    pltpu.make_async_copy(gflat.at[ibufb.at[1, 0]], buf1, semg1).wait()
    pltpu.sync_copy(buf1, acc_sh.at[ibufb.at[1, 1]], add=True)
    pltpu.make_async_copy(gflat.at[ibufa.at[0, 0]], buf0, semg0).wait()
    pltpu.sync_copy(buf0, acc_sh.at[ibufa.at[0, 1]], add=True)

    plsc.subcore_barrier()

    @pl.when(s < NS - 1)
    def _():
        pltpu.sync_copy(acc_sh.at[pl.ds(s * 640, 640)],
                        acc_hbm.at[c].at[pl.ds(s * 640, 640)])

    @pl.when(s == NS - 1)
    def _():
        pltpu.sync_copy(acc_sh.at[pl.ds(9600, 400)],
                        acc_hbm.at[c].at[pl.ds(9600, 400)])


def _tc_dinv_body(hist_ref, dinv_ref):
    deg = jnp.sum(hist_ref[...], axis=1) + 1.0
    dinv_ref[...] = lax.rsqrt(deg)[:, :N, None]


_tc_dinv = pl.pallas_call(
    _tc_dinv_body,
    out_shape=jax.ShapeDtypeStruct((NC, N, 1), jnp.float32),
)


def _tc_layer1_body(x_ref, w0_ref, w1_ref, dinv_ref, g_ref):
    dinv = dinv_ref[...]
    xb = x_ref[...]
    h0 = jnp.dot(xb, w0_ref[...], preferred_element_type=jnp.float32)
    h1 = jnp.dot(xb, w1_ref[...], preferred_element_type=jnp.float32)
    g_ref[0] = h0 * dinv[0]
    g_ref[1] = h1 * dinv[1]


def _tc_layer2_body(acc_ref, dinv_ref, b1_ref, w0_ref, w1_ref, g_ref):
    dinv = dinv_ref[...]
    h = jax.nn.relu(acc_ref[0] * dinv[0] + b1_ref[0]
                    + acc_ref[1] * dinv[1] + b1_ref[1])
    h0 = jnp.dot(h, w0_ref[...], preferred_element_type=jnp.float32)
    h1 = jnp.dot(h, w1_ref[...], preferred_element_type=jnp.float32)
    g_ref[0] = h0 * dinv[0]
    g_ref[1] = h1 * dinv[1]


def _tc_final_body(acc_ref, dinv_ref, b2_ref, out_ref):
    dinv = dinv_ref[...]
    out_ref[...] = (acc_ref[0] * dinv[0] + b2_ref[0]
                    + acc_ref[1] * dinv[1] + b2_ref[1])


_w_spec = pl.BlockSpec((D, D), lambda i: (0, 0))
_b_spec = pl.BlockSpec((NC, 1, D), lambda i: (0, 0, 0))
_g_spec = pl.BlockSpec((NC, RBLK, D), lambda i: (0, i, 0))
_dinv_spec = pl.BlockSpec((NC, RBLK, 1), lambda i: (0, i, 0))
_x_spec = pl.BlockSpec((RBLK, D), lambda i: (i, 0))

_tc_layer1 = pl.pallas_call(
    _tc_layer1_body,
    grid=(GRID,),
    in_specs=[_x_spec, _w_spec, _w_spec, _dinv_spec],
    out_specs=_g_spec,
    out_shape=jax.ShapeDtypeStruct((NC, N, D), jnp.float32),
)

_tc_layer2 = pl.pallas_call(
    _tc_layer2_body,
    grid=(GRID,),
    in_specs=[_g_spec, _dinv_spec, _b_spec, _w_spec, _w_spec],
    out_specs=_g_spec,
    out_shape=jax.ShapeDtypeStruct((NC, N, D), jnp.float32),
)

_tc_final = pl.pallas_call(
    _tc_final_body,
    grid=(GRID,),
    in_specs=[_g_spec, _dinv_spec, _b_spec],
    out_specs=_x_spec,
    out_shape=jax.ShapeDtypeStruct((N, D), jnp.float32),
)


def _prep_indices(ei):
    r = ei[0].astype(jnp.int32).reshape(NS, EPT)
    c = ei[1].astype(jnp.int32).reshape(NS, EPT)
    pad = ((0, 0), (0, EPAD - EPT))
    r = jnp.pad(r, pad, constant_values=0).reshape(NS, NCH, CHUNK)
    c = jnp.pad(c, pad, constant_values=N).reshape(NS, NCH, CHUNK)
    return jnp.stack([r, c], axis=2)


@jax.jit
def kernel(x, edge_index_0, edge_index_1,
           W1_0, b1_0, W1_1, b1_1, W2_0, b2_0, W2_1, b2_1):
    dsts = jnp.stack([edge_index_0[1].astype(jnp.int32),
                      edge_index_1[1].astype(jnp.int32)])
    hist = _sc_degree(dsts)

    idx = jnp.stack([_prep_indices(edge_index_0),
                     _prep_indices(edge_index_1)])
    b1 = jnp.stack([b1_0, b1_1]).reshape(NC, 1, D)
    b2 = jnp.stack([b2_0, b2_1]).reshape(NC, 1, D)

    dinv = _tc_dinv(hist)
    g1 = _tc_layer1(x, W1_0, W1_1, dinv)
    acc1 = _sc_scatter(g1, idx)
    g2 = _tc_layer2(acc1, dinv, b1, W2_0, W2_1)
    acc2 = _sc_scatter(g2, idx)
    return _tc_final(acc2, dinv, b2)


# speedup vs baseline: 1.7013x; 1.0060x over previous
import functools

import jax
import jax.numpy as jnp
from jax import lax
from jax.experimental import pallas as pl
from jax.experimental.pallas import tpu as pltpu
from jax.experimental.pallas import tpu_sc as plsc

N = 10000
E = 320000
D = 128
NC = 2
NS = 16
EPT = E // NS
CHUNK = 128
NCH = EPT // CHUNK + 1
EPAD = NCH * CHUNK
RBLK = 1000
GRID = N // RBLK
HPAD = ((N + 1 + 15) // 16) * 16


_sc_mesh = plsc.VectorSubcoreMesh(core_axis_name="c", subcore_axis_name="s")


SLAB = 19968

@functools.partial(
    pl.kernel,
    out_type=jax.ShapeDtypeStruct((NC, NS, HPAD), jnp.float32),
    mesh=_sc_mesh,
    scratch_types=[
        pltpu.VMEM((SLAB,), jnp.int32),
        pltpu.VMEM((HPAD,), jnp.float32),
    ],
    compiler_params=pltpu.CompilerParams(needs_layout_passes=False),
)
def _sc_degree(e0_hbm, e1_hbm, hist_hbm, col_v, hist_v):
    c = lax.axis_index("c")
    s = lax.axis_index("s")
    base = pl.multiple_of(s * SLAB, 128)

    @pl.when(c == 0)
    def _():
        pltpu.sync_copy(e0_hbm.at[pl.ds(base, SLAB)], col_v)

    @pl.when(c == 1)
    def _():
        pltpu.sync_copy(e1_hbm.at[pl.ds(base, SLAB)], col_v)

    zeros16 = jnp.zeros((16,), jnp.float32)

    def zbody(i, _):
        hist_v[pl.ds(i * 16, 16)] = zeros16
        return ()

    lax.fori_loop(0, HPAD // 16, zbody, (), unroll=8)

    ones16 = jnp.ones((16,), jnp.float32)

    def hbody(i, _):
        idx = col_v[pl.ds(i * 16, 16)]
        plsc.addupdate_scatter(hist_v, [idx], ones16)
        return ()

    lax.fori_loop(0, SLAB // 16, hbody, (), unroll=8)

    @pl.when(s < 4)
    def _():
        xb = pl.multiple_of(NS * SLAB + s * CHUNK, 128)

        @pl.when(c == 0)
        def _():
            pltpu.sync_copy(e0_hbm.at[pl.ds(xb, CHUNK)],
                            col_v.at[pl.ds(0, CHUNK)])

        @pl.when(c == 1)
        def _():
            pltpu.sync_copy(e1_hbm.at[pl.ds(xb, CHUNK)],
                            col_v.at[pl.ds(0, CHUNK)])

        def xbody(i, _):
            idx = col_v[pl.ds(i * 16, 16)]
            plsc.addupdate_scatter(hist_v, [idx], ones16)
            return ()

        lax.fori_loop(0, CHUNK // 16, xbody, ())

    pltpu.sync_copy(hist_v, hist_hbm.at[c, s])


@functools.partial(
    pl.kernel,
    out_type=jax.ShapeDtypeStruct((NC, N, D), jnp.float32),
    mesh=_sc_mesh,
    scratch_types=[
        pltpu.VMEM((2, 2, CHUNK), jnp.int32),
        pltpu.VMEM((2, 2, CHUNK), jnp.int32),
        pltpu.VMEM((CHUNK, D), jnp.float32),
        pltpu.VMEM((CHUNK, D), jnp.float32),
        pltpu.VMEM_SHARED((N + 8, D), jnp.float32),
        pltpu.SemaphoreType.DMA,
        pltpu.SemaphoreType.DMA,
        pltpu.SemaphoreType.DMA,
        pltpu.SemaphoreType.DMA,
    ],
)
def _sc_scatter(g_hbm, idx_hbm, acc_hbm,
                ibufa, ibufb, buf0, buf1, acc_sh, semia, semib, semg0, semg1):
    c = lax.axis_index("c")
    s = lax.axis_index("s")
    gflat = g_hbm.at[c]
    myidx = idx_hbm.at[c, s]

    def idx_pair(j):
        return myidx.at[pl.ds(j, 2)]

    @pl.when(s < NS - 1)
    def _():
        pltpu.sync_copy(gflat.at[pl.ds(s * 640, 640)],
                        acc_sh.at[pl.ds(s * 640, 640)])

    @pl.when(s == NS - 1)
    def _():
        pltpu.sync_copy(gflat.at[pl.ds(9600, 400)],
                        acc_sh.at[pl.ds(9600, 400)])

    plsc.subcore_barrier()

    pltpu.sync_copy(idx_pair(0), ibufa)
    pltpu.async_copy(idx_pair(2), ibufb, semib)
    pltpu.async_copy(gflat.at[ibufa.at[0, 0]], buf0, semg0)

    def body(kk, _):
        j0 = 4 * kk
        pltpu.make_async_copy(gflat.at[ibufa.at[0, 0]], buf0, semg0).wait()
        pltpu.async_copy(gflat.at[ibufa.at[1, 0]], buf1, semg1)
        pltpu.sync_copy(buf0, acc_sh.at[ibufa.at[0, 1]], add=True)
        pltpu.make_async_copy(idx_pair(j0 + 2), ibufb, semib).wait()
        pltpu.make_async_copy(gflat.at[ibufa.at[1, 0]], buf1, semg1).wait()
        pltpu.async_copy(gflat.at[ibufb.at[0, 0]], buf0, semg0)
        pltpu.sync_copy(buf1, acc_sh.at[ibufa.at[1, 1]], add=True)
        pltpu.async_copy(idx_pair(j0 + 4), ibufa, semia)
        pltpu.make_async_copy(gflat.at[ibufb.at[0, 0]], buf0, semg0).wait()
        pltpu.async_copy(gflat.at[ibufb.at[1, 0]], buf1, semg1)
        pltpu.sync_copy(buf0, acc_sh.at[ibufb.at[0, 1]], add=True)
        pltpu.make_async_copy(idx_pair(j0 + 4), ibufa, semia).wait()
        pltpu.make_async_copy(gflat.at[ibufb.at[1, 0]], buf1, semg1).wait()
        pltpu.async_copy(gflat.at[ibufa.at[0, 0]], buf0, semg0)
        pltpu.sync_copy(buf1, acc_sh.at[ibufb.at[1, 1]], add=True)
        pltpu.async_copy(idx_pair(j0 + 6), ibufb, semib)
        return ()

    lax.fori_loop(0, (NCH - 5) // 4, body, ())

    pltpu.make_async_copy(gflat.at[ibufa.at[0, 0]], buf0, semg0).wait()
    pltpu.async_copy(gflat.at[ibufa.at[1, 0]], buf1, semg1)
    pltpu.sync_copy(buf0, acc_sh.at[ibufa.at[0, 1]], add=True)
    pltpu.make_async_copy(idx_pair(NCH - 3), ibufb, semib).wait()
    pltpu.make_async_copy(gflat.at[ibufa.at[1, 0]], buf1, semg1).wait()
    pltpu.async_copy(gflat.at[ibufb.at[0, 0]], buf0, semg0)
    pltpu.sync_copy(buf1, acc_sh.at[ibufa.at[1, 1]], add=True)
    pltpu.sync_copy(myidx.at[NCH - 1], ibufa.at[0])
    pltpu.make_async_copy(gflat.at[ibufb.at[0, 0]], buf0, semg0).wait()
    pltpu.async_copy(gflat.at[ibufb.at[1, 0]], buf1, semg1)
    pltpu.sync_copy(buf0, acc_sh.at[ibufb.at[0, 1]], add=True)
    pltpu.async_copy(gflat.at[ibufa.at[0, 0]], buf0, semg0)
    pltpu.make_async_copy(gflat.at[ibufb.at[1, 0]], buf1, semg1).wait()
    pltpu.sync_copy(buf1, acc_sh.at[ibufb.at[1, 1]], add=True)
    pltpu.make_async_copy(gflat.at[ibufa.at[0, 0]], buf0, semg0).wait()
    pltpu.sync_copy(buf0, acc_sh.at[ibufa.at[0, 1]], add=True)

    plsc.subcore_barrier()

    @pl.when(s < NS - 1)
    def _():
        pltpu.sync_copy(acc_sh.at[pl.ds(s * 640, 640)],
                        acc_hbm.at[c].at[pl.ds(s * 640, 640)])

    @pl.when(s == NS - 1)
    def _():
        pltpu.sync_copy(acc_sh.at[pl.ds(9600, 400)],
                        acc_hbm.at[c].at[pl.ds(9600, 400)])


def _tc_dinv_body(hist_ref, dinv_ref):
    deg = jnp.sum(hist_ref[...], axis=1) + 1.0
    dinv_ref[...] = lax.rsqrt(deg)[:, :N, None]


_tc_dinv = pl.pallas_call(
    _tc_dinv_body,
    out_shape=jax.ShapeDtypeStruct((NC, N, 1), jnp.float32),
)


def _tc_layer1_body(x_ref, w0_ref, w1_ref, dinv_ref, g_ref):
    dinv = dinv_ref[...]
    xb = x_ref[...]
    h0 = jnp.dot(xb, w0_ref[...], preferred_element_type=jnp.float32)
    h1 = jnp.dot(xb, w1_ref[...], preferred_element_type=jnp.float32)
    g_ref[0] = h0 * dinv[0]
    g_ref[1] = h1 * dinv[1]


def _tc_layer2_body(acc_ref, dinv_ref, b1_ref, w0_ref, w1_ref, g_ref):
    dinv = dinv_ref[...]
    h = jax.nn.relu(acc_ref[0] * dinv[0] + b1_ref[0]
                    + acc_ref[1] * dinv[1] + b1_ref[1])
    h0 = jnp.dot(h, w0_ref[...], preferred_element_type=jnp.float32)
    h1 = jnp.dot(h, w1_ref[...], preferred_element_type=jnp.float32)
    g_ref[0] = h0 * dinv[0]
    g_ref[1] = h1 * dinv[1]


def _tc_final_body(acc_ref, dinv_ref, b2_ref, out_ref):
    dinv = dinv_ref[...]
    out_ref[...] = (acc_ref[0] * dinv[0] + b2_ref[0]
                    + acc_ref[1] * dinv[1] + b2_ref[1])


_w_spec = pl.BlockSpec((D, D), lambda i: (0, 0))
_b_spec = pl.BlockSpec((NC, 1, D), lambda i: (0, 0, 0))
_g_spec = pl.BlockSpec((NC, RBLK, D), lambda i: (0, i, 0))
_dinv_spec = pl.BlockSpec((NC, RBLK, 1), lambda i: (0, i, 0))
_x_spec = pl.BlockSpec((RBLK, D), lambda i: (i, 0))

_tc_layer1 = pl.pallas_call(
    _tc_layer1_body,
    grid=(GRID,),
    in_specs=[_x_spec, _w_spec, _w_spec, _dinv_spec],
    out_specs=_g_spec,
    out_shape=jax.ShapeDtypeStruct((NC, N, D), jnp.float32),
)

_tc_layer2 = pl.pallas_call(
    _tc_layer2_body,
    grid=(GRID,),
    in_specs=[_g_spec, _dinv_spec, _b_spec, _w_spec, _w_spec],
    out_specs=_g_spec,
    out_shape=jax.ShapeDtypeStruct((NC, N, D), jnp.float32),
)

_tc_final = pl.pallas_call(
    _tc_final_body,
    grid=(GRID,),
    in_specs=[_g_spec, _dinv_spec, _b_spec],
    out_specs=_x_spec,
    out_shape=jax.ShapeDtypeStruct((N, D), jnp.float32),
)


def _prep_indices(ei):
    r = ei[0].astype(jnp.int32).reshape(NS, EPT)
    c = ei[1].astype(jnp.int32).reshape(NS, EPT)
    pad = ((0, 0), (0, EPAD - EPT))
    r = jnp.pad(r, pad, constant_values=0).reshape(NS, NCH, CHUNK)
    c = jnp.pad(c, pad, constant_values=N).reshape(NS, NCH, CHUNK)
    return jnp.stack([r, c], axis=2)


@jax.jit
def kernel(x, edge_index_0, edge_index_1,
           W1_0, b1_0, W1_1, b1_1, W2_0, b2_0, W2_1, b2_1):
    hist = _sc_degree(edge_index_0[1].astype(jnp.int32),
                      edge_index_1[1].astype(jnp.int32))

    idx = jnp.stack([_prep_indices(edge_index_0),
                     _prep_indices(edge_index_1)])
    b1 = jnp.stack([b1_0, b1_1]).reshape(NC, 1, D)
    b2 = jnp.stack([b2_0, b2_1]).reshape(NC, 1, D)

    dinv = _tc_dinv(hist)
    g1 = _tc_layer1(x, W1_0, W1_1, dinv)
    acc1 = _sc_scatter(g1, idx)
    g2 = _tc_layer2(acc1, dinv, b1, W2_0, W2_1)
    acc2 = _sc_scatter(g2, idx)
    return _tc_final(acc2, dinv, b2)
